# baseline (speedup 1.0000x reference)
import jax
import jax.numpy as jnp
from jax.experimental import pallas as pl
from jax.experimental.pallas import tpu as pltpu

B = 4
H = 76
W = 76
A = 9
M = 50
C = 80
PAD = 608.0
HW = H * W
N = HW * A
IGNORE_IOU = 0.7

LB = 4096
NBL = -(-N // LB)


def _body(boxt_ref, gt_ref, anct_ref, lab_ref,
          obj_ref, cen_ref, sca_ref, wei_ref, cls_ref):
    i = pl.program_id(1)

    boxt = boxt_ref[0]
    gt = gt_ref[0]
    anct = anct_ref[...]
    lab = lab_ref[0]

    gx0 = gt[:, 0:1]
    gy0 = gt[:, 1:2]
    gx1 = gt[:, 2:3]
    gy1 = gt[:, 3:4]
    gtx = (gx0 + gx1) * 0.5
    gty = (gy0 + gy1) * 0.5
    gtw = gx1 - gx0
    gth = gy1 - gy0

    aw = anct[0:1, :]
    ah = anct[1:2, :]
    tlx = jnp.maximum(-0.5 * aw, -0.5 * gtw)
    tly = jnp.maximum(-0.5 * ah, -0.5 * gth)
    brx = jnp.minimum(0.5 * aw, 0.5 * gtw)
    bry = jnp.minimum(0.5 * ah, 0.5 * gth)
    iw = jnp.maximum(brx - tlx, 0.0)
    ih = jnp.maximum(bry - tly, 0.0)
    inter = iw * ih
    area_a = (0.5 * aw - (-0.5 * aw)) * (0.5 * ah - (-0.5 * ah))
    area_g = (0.5 * gtw - (-0.5 * gtw)) * (0.5 * gth - (-0.5 * gth))
    iou_am = inter / (area_a + area_g - inter + 1e-12)
    maxv = jnp.max(iou_am, axis=1, keepdims=True)
    a_iota = jax.lax.broadcasted_iota(jnp.int32, (M, 9), 1)
    match = jnp.min(jnp.where(iou_am == maxv, a_iota, 9),
                    axis=1, keepdims=True)
    amask = a_iota == match
    awm = jnp.sum(jnp.where(amask, jnp.broadcast_to(aw, (M, 9)), 0.0),
                  axis=1, keepdims=True)
    ahm = jnp.sum(jnp.where(amask, jnp.broadcast_to(ah, (M, 9)), 0.0),
                  axis=1, keepdims=True)

    valid = (gx0 >= 0.0) & (gy0 >= 0.0) & (gx1 >= 0.0) & (gy1 >= 0.0)
    loc_x = jnp.clip((gtx / PAD * W).astype(jnp.int32), 0, W - 1)
    loc_y = jnp.clip((gty / PAD * H).astype(jnp.int32), 0, H - 1)
    index = jnp.where(valid, loc_y * W + loc_x, HW)
    row = index * A + match
    tx = gtx / PAD * W - loc_x.astype(jnp.float32)
    ty = gty / PAD * H - loc_y.astype(jnp.float32)
    sw = jnp.log(jnp.maximum(gtw, 1.0) / awm)
    sh = jnp.log(jnp.maximum(gth, 1.0) / ahm)
    wgt = 2.0 - gtw * gth / PAD / PAD
    c_iota = jax.lax.broadcasted_iota(jnp.int32, (M, C + 1), 1)
    lmat = ((lab - 1) == c_iota).astype(jnp.float32) + \
        (c_iota == C).astype(jnp.float32)

    ridx = i * LB + jax.lax.broadcasted_iota(jnp.int32, (1, LB), 1)
    eq = row == ridx
    eqf = eq.astype(jnp.float32)
    m_iota = jax.lax.broadcasted_iota(jnp.int32, (M, 1), 0)
    win = jnp.max(jnp.where(eq, jnp.broadcast_to(m_iota, (M, LB)), -1),
                  axis=0, keepdims=True)
    hit = win >= 0
    ohwf = (m_iota == win).astype(jnp.float32)
    txv = jnp.sum(ohwf * tx, axis=0, keepdims=True)
    tyv = jnp.sum(ohwf * ty, axis=0, keepdims=True)
    swv = jnp.sum(ohwf * sw, axis=0, keepdims=True)
    shv = jnp.sum(ohwf * sh, axis=0, keepdims=True)
    wgv = jnp.sum(ohwf * wgt, axis=0, keepdims=True)

    dims = (((0,), (0,)), ((), ()))
    counts = jax.lax.dot_general(eqf, lmat, dims,
                                 preferred_element_type=jnp.float32)
    anyeq = counts[:, C:C + 1] > 0.5
    cls = jnp.where(anyeq, jnp.minimum(counts[:, :C], 1.0), -1.0)

    px0 = boxt[0:1, :]
    py0 = boxt[1:2, :]
    px1 = boxt[2:3, :]
    py1 = boxt[3:4, :]
    itlx = jnp.maximum(px0, gx0)
    itly = jnp.maximum(py0, gy0)
    ibrx = jnp.minimum(px1, gx1)
    ibry = jnp.minimum(py1, gy1)
    iiw = jnp.maximum(ibrx - itlx, 0.0)
    iih = jnp.maximum(ibry - itly, 0.0)
    pinter = iiw * iih
    parea = (px1 - px0) * (py1 - py0)
    garea = (gx1 - gx0) * (gy1 - gy0)
    piou = pinter / (parea + garea - pinter + 1e-12)
    pmax = jnp.max(piou, axis=0, keepdims=True)
    dyn = jnp.where(pmax > IGNORE_IOU, -1.0, 0.0)

    obj_ref[0] = jnp.where(hit, 1.0, dyn)
    cen_ref[0] = jnp.where(hit, jnp.concatenate([txv, tyv], axis=0), 0.0)
    sca_ref[0] = jnp.where(hit, jnp.concatenate([swv, shv], axis=0), 0.0)
    wei_ref[0] = jnp.where(hit, jnp.concatenate([wgv, wgv], axis=0), 0.0)
    cls_ref[0] = cls


def kernel(box_preds, gt_boxes, anchors, gt_labels):
    box_t = jnp.transpose(box_preds, (0, 2, 1))
    anc_t = jnp.transpose(anchors, (1, 0))
    lab = gt_labels.reshape(B, M, 1)

    grid = (B, NBL)
    out = pl.pallas_call(
        _body,
        grid=grid,
        in_specs=[
            pl.BlockSpec((1, 4, LB), lambda b, i: (b, 0, i)),
            pl.BlockSpec((1, M, 4), lambda b, i: (b, 0, 0)),
            pl.BlockSpec((2, 9), lambda b, i: (0, 0)),
            pl.BlockSpec((1, M, 1), lambda b, i: (b, 0, 0)),
        ],
        out_specs=[
            pl.BlockSpec((1, 1, LB), lambda b, i: (b, 0, i)),
            pl.BlockSpec((1, 2, LB), lambda b, i: (b, 0, i)),
            pl.BlockSpec((1, 2, LB), lambda b, i: (b, 0, i)),
            pl.BlockSpec((1, 2, LB), lambda b, i: (b, 0, i)),
            pl.BlockSpec((1, LB, C), lambda b, i: (b, i, 0)),
        ],
        out_shape=[
            jax.ShapeDtypeStruct((B, 1, N), jnp.float32),
            jax.ShapeDtypeStruct((B, 2, N), jnp.float32),
            jax.ShapeDtypeStruct((B, 2, N), jnp.float32),
            jax.ShapeDtypeStruct((B, 2, N), jnp.float32),
            jax.ShapeDtypeStruct((B, N, C), jnp.float32),
        ],
        compiler_params=pltpu.CompilerParams(
            dimension_semantics=("parallel", "parallel"),
        ),
    )(box_t, gt_boxes, anc_t, lab)
    obj, cen, sca, wei, cls = out
    tr = lambda x: jnp.transpose(x, (0, 2, 1))
    return (obj.reshape(B, N, 1), tr(cen), tr(sca), tr(wei), cls)
